# baseline (speedup 1.0000x reference)
import math

import jax
import jax.numpy as jnp
from jax import lax
from jax.experimental import pallas as pl
from jax.experimental.pallas import tpu as pltpu

TOTAL_LATENT = 97
PAD_LATENT = 128
K_SORT = 30
C1, C2 = 16, 32
KW2 = 5
OUT_DIM = 10
F_IN = 8
H = 48
RATIOS = [0.9, 0.7, 0.6, 0.5]
DEPTH = len(RATIOS)
KP = (K_SORT - 2) // 2 + 1
LOUT = KP - KW2 + 1


def _ladder(n):
    out = [n]
    for r in RATIOS:
        out.append(int(math.ceil(r * out[-1])))
    return out


def _make_unet_kernel(lad):
    def unet_kernel(a_ref, x_ref, dw0_ref, db0_ref, dw_ref, db_ref, pw_ref,
                    uw_ref, ub_ref, uwl_ref, ubl_ref,
                    w1_ref, b1_ref, w2_ref, b2_ref, wd_ref, bd_ref, o_ref):
        f32 = jnp.float32
        _eyes, _ltm, _fiotas = {}, {}, {}

        def fiota(shape, dim):
            key = (shape, dim)
            if key not in _fiotas:
                _fiotas[key] = lax.broadcasted_iota(
                    jnp.int32, shape, dim).astype(jnp.float32)
            return _fiotas[key]

        def eye(n):
            if n not in _eyes:
                r = lax.broadcasted_iota(jnp.int32, (n, n), 0)
                c = lax.broadcasted_iota(jnp.int32, (n, n), 1)
                _eyes[n] = jnp.where(r == c, 1.0, 0.0)
            return _eyes[n]

        def lt_mask(n):
            if n not in _ltm:
                r = lax.broadcasted_iota(jnp.int32, (n, n), 0)
                c = lax.broadcasted_iota(jnp.int32, (n, n), 1)
                _ltm[n] = c < r
            return _ltm[n]

        def rowvec(v_col, n):
            diag = eye(n) * jnp.broadcast_to(v_col, (n, n))
            return jnp.dot(jnp.ones((1, n), f32), diag,
                           preferred_element_type=f32)


        def ranks(s_col, n):
            s_row = jnp.broadcast_to(rowvec(s_col, n), (n, n))
            s_cb = jnp.broadcast_to(s_col, (n, n))
            before = jnp.where((s_row > s_cb) | ((s_row == s_cb) & lt_mask(n)),
                               1.0, 0.0)
            return jnp.sum(before, axis=1, keepdims=True)

        def gcn_norm(A, n):
            ahat = A + 2.0 * eye(n)
            dr = lax.rsqrt(jnp.sum(ahat, axis=1, keepdims=True))
            return ahat, dr

        def gcn(norm, x, W, b, relu):
            ahat, dr = norm
            xw = jnp.dot(x, W, preferred_element_type=f32)
            out = dr * jnp.dot(ahat, dr * xw, preferred_element_type=f32) + b
            return jnp.maximum(out, 0.0) if relu else out

        def sel_mat_t(rank, n, k):
            cc = fiota((n, k), 1)
            return jnp.where(jnp.broadcast_to(rank, (n, k)) == cc, 1.0, 0.0)

        def topk_augmented(x, A, w_col, n, k, binary_adj, defer_diag_zero):
            score = jnp.tanh(jnp.dot(x, w_col, preferred_element_type=f32))
            rank = ranks(score, n)
            rank_row = jnp.broadcast_to(rowvec(rank, n), (k, n))
            p = jnp.where(rank_row == fiota((k, n), 0), 1.0, 0.0)
            x_new = jnp.dot(p, x * score, preferred_element_type=f32)
            e = eye(n)
            at = A + e
            pt = sel_mat_t(rank, n, k)
            if binary_adj:
                atb = at.astype(jnp.bfloat16)
                a_aug = jnp.dot(atb, atb, preferred_element_type=f32)
            else:
                a_aug = jnp.dot(at, at, preferred_element_type=f32)
            if not defer_diag_zero:
                a_aug = a_aug * (1.0 - e)
            pa = jnp.dot(p, a_aug, preferred_element_type=f32)
            a_new = jnp.dot(pa, pt, preferred_element_type=f32)
            if defer_diag_zero:
                a_new = a_new * (1.0 - eye(k))
            return x_new, a_new, rank, pt

        A = a_ref[...]
        norm = gcn_norm(A, lad[0])
        x = gcn(norm, x_ref[...], dw0_ref[...], db0_ref[...], True)
        xs, norms, pts = [x], [norm], []
        for i in range(DEPTH):
            n, k = lad[i], lad[i + 1]
            x, A, rank, pt = topk_augmented(x, A, pw_ref[:, i:i + 1], n, k,
                                            i == 0, i < 2)
            norm = gcn_norm(A, k)
            x = gcn(norm, x, dw_ref[i], db_ref[i], True)
            if i < DEPTH - 1:
                xs.append(x)
                norms.append(norm)
            pts.append(pt)

        for i in range(DEPTH):
            j = DEPTH - 1 - i
            up = jnp.dot(pts[j], x, preferred_element_type=f32)
            x = xs[j] + up
            if i < DEPTH - 1:
                x = gcn(norms[j], x, uw_ref[i], ub_ref[i], True)
            else:
                x = gcn(norms[j], x, uwl_ref[...], ubl_ref[...], False)

        n = lad[0]
        srt = ranks(x[:, TOTAL_LATENT - 1:TOTAL_LATENT], n)
        rank_row = jnp.broadcast_to(rowvec(srt, n), (KP, n))
        rr = fiota((KP, n), 0)
        p_even = jnp.where(rank_row == 2 * rr, 1.0, 0.0)
        p_odd = jnp.where(rank_row == 2 * rr + 1, 1.0, 0.0)
        xe = jnp.dot(p_even, x, preferred_element_type=f32)
        xo = jnp.dot(p_odd, x, preferred_element_type=f32)

        he = jnp.dot(xe, w1_ref[...], preferred_element_type=f32) + b1_ref[...]
        ho = jnp.dot(xo, w1_ref[...], preferred_element_type=f32) + b1_ref[...]
        hp = jnp.maximum(jnp.maximum(he, ho), 0.0)
        cols = jnp.concatenate([hp[j:j + LOUT, :] for j in range(KW2)], axis=1)
        h2 = jnp.maximum(
            jnp.dot(cols, w2_ref[...], preferred_element_type=f32)
            + b2_ref[...], 0.0)
        out = bd_ref[...]
        for t in range(LOUT):
            out = out + jnp.dot(h2[t:t + 1, :], wd_ref[t],
                                preferred_element_type=f32)
        o_ref[...] = jnp.maximum(out, 0.0)

    return unet_kernel


def _graph_call(params, pw, x, A):
    k = _make_unet_kernel(_ladder(A.shape[0]))
    return pl.pallas_call(
        k, out_shape=jax.ShapeDtypeStruct((1, OUT_DIM), jnp.float32),
    )(A, x, params['down_w0'], params['down_b0'], params['down_w'],
      params['down_b'], pw, params['up_w'], params['up_b'],
      params['up_w_last'], params['up_b_last'], params['w1'], params['b1'],
      params['w2'], params['b2'], params['wd'], params['bd'])


def kernel(down_w0, down_b0, down_w, down_b, pool_w, up_w, up_b, up_w_last,
           up_b_last, w1, b1, w2, b2, wd, bd, x0, A0, x1, A1):
    params = {
        'down_w0': down_w0, 'down_b0': down_b0,
        'down_w': down_w, 'down_b': down_b,
        'up_w': up_w, 'up_b': up_b,
        'up_w_last': up_w_last, 'up_b_last': up_b_last,
        'w1': w1, 'b1': b1, 'w2': w2, 'b2': b2, 'wd': wd, 'bd': bd,
    }
    pw = pool_w / jnp.sqrt(jnp.sum(pool_w * pool_w, axis=0, keepdims=True))
    o0 = _graph_call(params, pw, x0, A0)
    o1 = _graph_call(params, pw, x1, A1)
    return jnp.concatenate([o0, o1], axis=0)


# speedup vs baseline: 1.0008x; 1.0008x over previous
import math

import jax
import jax.numpy as jnp
from jax import lax
from jax.experimental import pallas as pl
from jax.experimental.pallas import tpu as pltpu

TOTAL_LATENT = 97
PAD_LATENT = 128
K_SORT = 30
C1, C2 = 16, 32
KW2 = 5
OUT_DIM = 10
F_IN = 8
H = 48
RATIOS = [0.9, 0.7, 0.6, 0.5]
DEPTH = len(RATIOS)
KP = (K_SORT - 2) // 2 + 1
LOUT = KP - KW2 + 1


def _ladder(n):
    out = [n]
    for r in RATIOS:
        out.append(int(math.ceil(r * out[-1])))
    return out


def _make_unet_kernel(lad):
    def unet_kernel(a_ref, x_ref, dw0_ref, db0_ref, dw_ref, db_ref, pw_ref,
                    uw_ref, ub_ref, uwl_ref, ubl_ref,
                    w1_ref, b1_ref, w2_ref, b2_ref, wd_ref, bd_ref, o_ref):
        f32 = jnp.float32
        _eyes, _ltm, _fiotas = {}, {}, {}

        def fiota(shape, dim):
            key = (shape, dim)
            if key not in _fiotas:
                _fiotas[key] = lax.broadcasted_iota(
                    jnp.int32, shape, dim).astype(jnp.float32)
            return _fiotas[key]

        def eye(n):
            if n not in _eyes:
                r = lax.broadcasted_iota(jnp.int32, (n, n), 0)
                c = lax.broadcasted_iota(jnp.int32, (n, n), 1)
                _eyes[n] = jnp.where(r == c, 1.0, 0.0)
            return _eyes[n]

        def lt_mask(n):
            if n not in _ltm:
                r = lax.broadcasted_iota(jnp.int32, (n, n), 0)
                c = lax.broadcasted_iota(jnp.int32, (n, n), 1)
                _ltm[n] = c < r
            return _ltm[n]

        def rowvec(v_col, n):
            diag = eye(n) * jnp.broadcast_to(v_col, (n, n))
            return jnp.dot(jnp.ones((1, n), f32), diag,
                           preferred_element_type=f32)


        def ranks(s_col, n):
            s_row = jnp.broadcast_to(rowvec(s_col, n), (n, n))
            s_cb = jnp.broadcast_to(s_col, (n, n))
            before = jnp.where((s_row > s_cb) | ((s_row == s_cb) & lt_mask(n)),
                               1.0, 0.0)
            return jnp.sum(before, axis=1, keepdims=True)

        def gcn_norm(A, n):
            ahat = A + 2.0 * eye(n)
            dr = lax.rsqrt(jnp.sum(ahat, axis=1, keepdims=True))
            return ahat, dr

        def gcn(norm, x, W, b, relu):
            ahat, dr = norm
            xw = jnp.dot(x, W, preferred_element_type=f32)
            out = dr * jnp.dot(ahat, dr * xw, preferred_element_type=f32) + b
            return jnp.maximum(out, 0.0) if relu else out

        def sel_mat_t(rank, n, k):
            cc = fiota((n, k), 1)
            return jnp.where(jnp.broadcast_to(rank, (n, k)) == cc, 1.0, 0.0)

        def topk_augmented(x, A, w_col, n, k, binary_adj):
            score = jnp.tanh(jnp.dot(x, w_col, preferred_element_type=f32))
            rank = ranks(score, n)
            rank_row = jnp.broadcast_to(rowvec(rank, n), (k, n))
            p = jnp.where(rank_row == fiota((k, n), 0), 1.0, 0.0)
            x_new = jnp.dot(p, x * score, preferred_element_type=f32)
            e = eye(n)
            at = A + e
            pt = sel_mat_t(rank, n, k)
            if binary_adj:
                atb = at.astype(jnp.bfloat16)
                a_aug = jnp.dot(atb, atb, preferred_element_type=f32)
            else:
                a_aug = jnp.dot(at, at, preferred_element_type=f32)
            a_aug = a_aug * (1.0 - e)
            pa = jnp.dot(p, a_aug, preferred_element_type=f32)
            a_new = jnp.dot(pa, pt, preferred_element_type=f32)
            return x_new, a_new, rank, pt

        A = a_ref[...]
        norm = gcn_norm(A, lad[0])
        x = gcn(norm, x_ref[...], dw0_ref[...], db0_ref[...], True)
        xs, norms, pts = [x], [norm], []
        for i in range(DEPTH):
            n, k = lad[i], lad[i + 1]
            x, A, rank, pt = topk_augmented(x, A, pw_ref[:, i:i + 1], n, k,
                                            i == 0)
            norm = gcn_norm(A, k)
            x = gcn(norm, x, dw_ref[i], db_ref[i], True)
            if i < DEPTH - 1:
                xs.append(x)
                norms.append(norm)
            pts.append(pt)

        for i in range(DEPTH):
            j = DEPTH - 1 - i
            up = jnp.dot(pts[j], x, preferred_element_type=f32)
            x = xs[j] + up
            if i < DEPTH - 1:
                x = gcn(norms[j], x, uw_ref[i], ub_ref[i], True)
            else:
                x = gcn(norms[j], x, uwl_ref[...], ubl_ref[...], False)

        n = lad[0]
        srt = ranks(x[:, TOTAL_LATENT - 1:TOTAL_LATENT], n)
        rank_row = jnp.broadcast_to(rowvec(srt, n), (KP, n))
        rr = fiota((KP, n), 0)
        p_even = jnp.where(rank_row == 2 * rr, 1.0, 0.0)
        p_odd = jnp.where(rank_row == 2 * rr + 1, 1.0, 0.0)
        xe = jnp.dot(p_even, x, preferred_element_type=f32)
        xo = jnp.dot(p_odd, x, preferred_element_type=f32)

        he = jnp.dot(xe, w1_ref[...], preferred_element_type=f32) + b1_ref[...]
        ho = jnp.dot(xo, w1_ref[...], preferred_element_type=f32) + b1_ref[...]
        hp = jnp.maximum(jnp.maximum(he, ho), 0.0)
        cols = jnp.concatenate([hp[j:j + LOUT, :] for j in range(KW2)], axis=1)
        h2 = jnp.maximum(
            jnp.dot(cols, w2_ref[...], preferred_element_type=f32)
            + b2_ref[...], 0.0)
        out = bd_ref[...]
        for t in range(LOUT):
            out = out + jnp.dot(h2[t:t + 1, :], wd_ref[t],
                                preferred_element_type=f32)
        o_ref[...] = jnp.maximum(out, 0.0)

    return unet_kernel


def _graph_call(params, pw, x, A):
    k = _make_unet_kernel(_ladder(A.shape[0]))
    return pl.pallas_call(
        k, out_shape=jax.ShapeDtypeStruct((1, OUT_DIM), jnp.float32),
    )(A, x, params['down_w0'], params['down_b0'], params['down_w'],
      params['down_b'], pw, params['up_w'], params['up_b'],
      params['up_w_last'], params['up_b_last'], params['w1'], params['b1'],
      params['w2'], params['b2'], params['wd'], params['bd'])


def kernel(down_w0, down_b0, down_w, down_b, pool_w, up_w, up_b, up_w_last,
           up_b_last, w1, b1, w2, b2, wd, bd, x0, A0, x1, A1):
    params = {
        'down_w0': down_w0, 'down_b0': down_b0,
        'down_w': down_w, 'down_b': down_b,
        'up_w': up_w, 'up_b': up_b,
        'up_w_last': up_w_last, 'up_b_last': up_b_last,
        'w1': w1, 'b1': b1, 'w2': w2, 'b2': b2, 'wd': wd, 'bd': bd,
    }
    pw = pool_w / jnp.sqrt(jnp.sum(pool_w * pool_w, axis=0, keepdims=True))
    o0 = _graph_call(params, pw, x0, A0)
    o1 = _graph_call(params, pw, x1, A1)
    return jnp.concatenate([o0, o1], axis=0)
